# baseline (speedup 1.0000x reference)
import functools

import jax
import jax.numpy as jnp
from jax import lax
from jax.experimental import pallas as pl
from jax.experimental.pallas import tpu as pltpu
from jax.experimental.pallas import tpu_sc as plsc

_D = 128
_NC = 2
_NS = 16
_NW = _NC * _NS
_CK = 128
_NPAD = 10240
_RB = 128
_NBLK = _NPAD // (_NS * _RB)
_ZB = 32
_R = 512


def _mesh():
    return plsc.VectorSubcoreMesh(core_axis_name="c", subcore_axis_name="s")


@functools.cache
def _make_deg(epad: int):
    ew = epad // _NW
    nchunk = ew // _CK
    grp = 8
    zper = _NPAD // _NS

    @functools.partial(
        pl.kernel,
        out_type=jax.ShapeDtypeStruct((_NC, _NPAD), jnp.float32),
        mesh=_mesh(),
        scratch_types=[
            pltpu.VMEM((nchunk, _CK), jnp.int32),
            pltpu.VMEM((_CK,), jnp.float32),
            pltpu.VMEM((zper,), jnp.float32),
            pltpu.VMEM_SHARED((_NPAD,), jnp.float32),
            pltpu.SemaphoreType.DMA,
        ],
    )
    def deg_kernel(dst_hbm, out_hbm, idx_all, ones_v, zbuf, acc, sem):
        cid = lax.axis_index("c")
        sid = lax.axis_index("s")
        wid = sid * _NC + cid
        pltpu.sync_copy(dst_hbm.at[wid], idx_all)
        ones16 = jnp.full((16,), 1.0, jnp.float32)
        zero16 = jnp.zeros((16,), jnp.float32)
        for j in range(_CK // 16):
            ones_v[pl.ds(j * 16, 16)] = ones16

        def zb(i, c):
            zbuf[pl.ds(i * 16, 16)] = zero16
            return c

        lax.fori_loop(0, zper // 16, zb, 0)
        pltpu.sync_copy(zbuf, acc.at[pl.ds(sid * zper, zper)])
        plsc.subcore_barrier()

        def body(g, carry):
            for b in range(grp):
                c = g * grp + b
                pltpu.async_copy(ones_v, acc.at[idx_all.at[c]], sem, add=True)
            for b in range(grp):
                pltpu.make_async_copy(ones_v, acc.at[idx_all.at[0]], sem).wait()
            return carry

        lax.fori_loop(0, nchunk // grp, body, 0)
        plsc.subcore_barrier()
        pltpu.sync_copy(acc.at[pl.ds(sid * zper, zper)],
                        out_hbm.at[cid, pl.ds(sid * zper, zper)])

    return deg_kernel


@functools.cache
def _make_agg(epad: int):
    ew = epad // _NW
    nchunk = ew // _CK

    @functools.partial(
        pl.kernel,
        out_type=jax.ShapeDtypeStruct((_NC, _NPAD, _D), jnp.float32),
        mesh=_mesh(),
        scratch_types=[
            pltpu.VMEM((2, _CK), jnp.int32),
            pltpu.VMEM((2, _CK), jnp.int32),
            pltpu.VMEM((_CK, _D), jnp.float32),
            pltpu.VMEM((_CK, _D), jnp.float32),
            pltpu.VMEM((_ZB, _D), jnp.float32),
            pltpu.VMEM_SHARED((_NPAD, _D), jnp.float32),
            pltpu.SemaphoreType.DMA,
            pltpu.SemaphoreType.DMA,
            pltpu.SemaphoreType.DMA,
            pltpu.SemaphoreType.DMA,
        ],
    )
    def agg_kernel(ed_hbm, u_hbm, out_hbm,
                   ib0, ib1, rows0, rows1, zbuf, acc,
                   gsem0, gsem1, isem0, isem1):
        cid = lax.axis_index("c")
        sid = lax.axis_index("s")
        wid = sid * _NC + cid
        zero16 = jnp.zeros((16,), jnp.float32)

        def zb(i, c):
            for j in range(_D // 16):
                zbuf[i, pl.ds(j * 16, 16)] = zero16
            return c

        lax.fori_loop(0, _ZB, zb, 0)
        row0 = sid * (_NBLK * _RB)

        def zcp(i, c):
            pltpu.sync_copy(zbuf, acc.at[pl.ds(row0 + i * _ZB, _ZB), :])
            return c

        lax.fori_loop(0, (_NBLK * _RB) // _ZB, zcp, 0)
        plsc.subcore_barrier()

        ib = (ib0, ib1)
        isems = (isem0, isem1)
        rows = (rows0, rows1)
        gsems = (gsem0, gsem1)
        pltpu.sync_copy(ed_hbm.at[wid, 0], ib0)
        pltpu.async_copy(ed_hbm.at[wid, 1], ib1, isem1)
        pltpu.async_copy(u_hbm.at[ib0.at[0]], rows0, gsem0)

        def body(g, carry):
            for b in range(2):
                c = 2 * g + b
                nb = 1 - b

                @pl.when(c + 1 < nchunk)
                def _():
                    pltpu.make_async_copy(ed_hbm.at[wid, c + 1],
                                          ib[nb], isems[nb]).wait()
                    pltpu.async_copy(u_hbm.at[ib[nb].at[0]],
                                     rows[nb], gsems[nb])

                pltpu.make_async_copy(u_hbm.at[ib[b].at[0]],
                                      rows[b], gsems[b]).wait()
                pltpu.sync_copy(rows[b], acc.at[ib[b].at[1]], add=True)

                @pl.when(c + 2 < nchunk)
                def _():
                    pltpu.async_copy(ed_hbm.at[wid, c + 2], ib[b], isems[b])
            return carry

        lax.fori_loop(0, nchunk // 2, body, 0)
        plsc.subcore_barrier()
        for b in range(_NBLK):
            r = row0 + b * _RB
            pltpu.sync_copy(acc.at[pl.ds(r, _RB), :],
                            out_hbm.at[cid, pl.ds(r, _RB), :])

    return agg_kernel


def _dinv_of(dp_ref):
    return lax.rsqrt(dp_ref[0] + dp_ref[1] + 1.0)


def _tc_scale_body(dp_ref, x_ref, u_ref):
    u_ref[...] = x_ref[...] * _dinv_of(dp_ref)


def _tc_hidden_body(dp_ref, s_ref, u1_ref, w_ref, b_ref, u2_ref):
    dinv = _dinv_of(dp_ref)
    ax = (s_ref[0] + s_ref[1] + u1_ref[...]) * dinv
    h = jnp.dot(ax, w_ref[...], preferred_element_type=jnp.float32) + b_ref[...]
    u2_ref[...] = jnp.maximum(h, 0.0) * dinv


def _tc_out_body(dp_ref, s_ref, u2_ref, wmu_ref, bmu_ref, wls_ref, bls_ref,
                 mu_ref, ls_ref):
    dinv = _dinv_of(dp_ref)
    ah = (s_ref[0] + s_ref[1] + u2_ref[...]) * dinv
    mu_ref[...] = jnp.dot(ah, wmu_ref[...],
                          preferred_element_type=jnp.float32) + bmu_ref[...]
    ls_ref[...] = jnp.dot(ah, wls_ref[...],
                          preferred_element_type=jnp.float32) + bls_ref[...]


def _dp_spec():
    return pl.BlockSpec((_NC, _R, 1), lambda i: (0, i, 0))


def _row_spec(d):
    return pl.BlockSpec((_R, d), lambda i: (i, 0))


def _s_spec():
    return pl.BlockSpec((_NC, _R, _D), lambda i: (0, i, 0))


def _full_spec(shape):
    return pl.BlockSpec(shape, lambda i: tuple(0 for _ in shape))


def kernel(x, edge_index, W1, b1, W_mu, b_mu, W_logstd, b_logstd):
    n, d = x.shape
    e = edge_index.shape[1]
    q = _NW * _CK * 8
    epad = q * (-(-e // q))
    npad_e = epad - e
    nchunk = epad // (_NW * _CK)
    src = jnp.concatenate(
        [edge_index[0], jnp.zeros((npad_e,), edge_index.dtype)])
    pad_dst = n + (jnp.arange(npad_e, dtype=edge_index.dtype) % (_NPAD - n))
    dst = jnp.concatenate([edge_index[1], pad_dst])
    src = src.reshape(_NW, nchunk, _CK)
    dst = dst.reshape(_NW, nchunk, _CK)
    ed = jnp.stack([src, dst], axis=2)

    degp = _make_deg(epad)(dst)
    dp3 = degp[:, :, None]

    grid = (_NPAD // _R,)
    u1 = pl.pallas_call(
        _tc_scale_body,
        grid=grid,
        in_specs=[_dp_spec(), _row_spec(d)],
        out_specs=_row_spec(d),
        out_shape=jax.ShapeDtypeStruct((n, d), jnp.float32),
    )(dp3, x)

    agg = _make_agg(epad)
    s1 = agg(ed, u1)

    dhid = W1.shape[1]
    u2 = pl.pallas_call(
        _tc_hidden_body,
        grid=grid,
        in_specs=[_dp_spec(), _s_spec(), _row_spec(d),
                  _full_spec(W1.shape), _full_spec((1, dhid))],
        out_specs=_row_spec(dhid),
        out_shape=jax.ShapeDtypeStruct((n, dhid), jnp.float32),
    )(dp3, s1, u1, W1, b1.reshape(1, -1))

    s2 = agg(ed, u2)

    dout = W_mu.shape[1]
    mu, logstd = pl.pallas_call(
        _tc_out_body,
        grid=grid,
        in_specs=[_dp_spec(), _s_spec(), _row_spec(dhid),
                  _full_spec(W_mu.shape), _full_spec((1, dout)),
                  _full_spec(W_logstd.shape), _full_spec((1, dout))],
        out_specs=[_row_spec(dout), _row_spec(dout)],
        out_shape=[jax.ShapeDtypeStruct((n, dout), jnp.float32),
                   jax.ShapeDtypeStruct((n, dout), jnp.float32)],
    )(dp3, s2, u2, W_mu, b_mu.reshape(1, -1), W_logstd, b_logstd.reshape(1, -1))

    return (mu, logstd)


# speedup vs baseline: 1.0526x; 1.0526x over previous
import functools

import jax
import jax.numpy as jnp
from jax import lax
from jax.experimental import pallas as pl
from jax.experimental.pallas import tpu as pltpu
from jax.experimental.pallas import tpu_sc as plsc

_D = 128
_NC = 2
_NS = 16
_NW = _NC * _NS
_CK = 128
_NPAD = 10240
_RB = 128
_NBLK = _NPAD // (_NS * _RB)
_ZB = 32
_R = 512


def _mesh():
    return plsc.VectorSubcoreMesh(core_axis_name="c", subcore_axis_name="s")


@functools.cache
def _make_deg(epad: int):
    ew = epad // _NW
    nchunk = ew // _CK
    grp = 8
    zper = _NPAD // _NS

    @functools.partial(
        pl.kernel,
        out_type=jax.ShapeDtypeStruct((_NC, _NPAD), jnp.float32),
        mesh=_mesh(),
        scratch_types=[
            pltpu.VMEM((nchunk, _CK), jnp.int32),
            pltpu.VMEM((_CK,), jnp.float32),
            pltpu.VMEM((zper,), jnp.float32),
            pltpu.VMEM_SHARED((_NPAD,), jnp.float32),
            pltpu.SemaphoreType.DMA,
        ],
    )
    def deg_kernel(dst_hbm, out_hbm, idx_all, ones_v, zbuf, acc, sem):
        cid = lax.axis_index("c")
        sid = lax.axis_index("s")
        wid = sid * _NC + cid
        pltpu.sync_copy(dst_hbm.at[wid], idx_all)
        ones16 = jnp.full((16,), 1.0, jnp.float32)
        zero16 = jnp.zeros((16,), jnp.float32)
        for j in range(_CK // 16):
            ones_v[pl.ds(j * 16, 16)] = ones16

        def zb(i, c):
            zbuf[pl.ds(i * 16, 16)] = zero16
            return c

        lax.fori_loop(0, zper // 16, zb, 0)
        pltpu.sync_copy(zbuf, acc.at[pl.ds(sid * zper, zper)])
        plsc.subcore_barrier()

        def body(g, carry):
            for b in range(grp):
                c = g * grp + b
                pltpu.async_copy(ones_v, acc.at[idx_all.at[c]], sem, add=True)
            for b in range(grp):
                pltpu.make_async_copy(ones_v, acc.at[idx_all.at[0]], sem).wait()
            return carry

        lax.fori_loop(0, nchunk // grp, body, 0)
        plsc.subcore_barrier()
        pltpu.sync_copy(acc.at[pl.ds(sid * zper, zper)],
                        out_hbm.at[cid, pl.ds(sid * zper, zper)])

    return deg_kernel


@functools.cache
def _make_agg(epad: int):
    tch = epad // _CK
    per_pair = tch // _NS
    f1 = max(2, (int(per_pair * 0.225) // 2) * 2)
    f0 = per_pair - f1

    @functools.partial(
        pl.kernel,
        out_type=jax.ShapeDtypeStruct((_NC, _NPAD, _D), jnp.float32),
        mesh=_mesh(),
        scratch_types=[
            pltpu.VMEM((2, _CK), jnp.int32),
            pltpu.VMEM((2, _CK), jnp.int32),
            pltpu.VMEM((_CK, _D), jnp.float32),
            pltpu.VMEM((_CK, _D), jnp.float32),
            pltpu.VMEM((_ZB, _D), jnp.float32),
            pltpu.VMEM_SHARED((_NPAD, _D), jnp.float32),
            pltpu.SemaphoreType.DMA,
            pltpu.SemaphoreType.DMA,
            pltpu.SemaphoreType.DMA,
            pltpu.SemaphoreType.DMA,
        ],
    )
    def agg_kernel(ed_hbm, u_hbm, out_hbm,
                   ib0, ib1, rows0, rows1, zbuf, acc,
                   gsem0, gsem1, isem0, isem1):
        cid = lax.axis_index("c")
        sid = lax.axis_index("s")
        start = lax.select(cid == 0, sid * f0, _NS * f0 + sid * f1)
        t = lax.select(cid == 0, jnp.int32(f0), jnp.int32(f1))
        zero16 = jnp.zeros((16,), jnp.float32)

        def zb(i, c):
            for j in range(_D // 16):
                zbuf[i, pl.ds(j * 16, 16)] = zero16
            return c

        lax.fori_loop(0, _ZB, zb, 0)
        row0 = sid * (_NBLK * _RB)

        def zcp(i, c):
            pltpu.sync_copy(zbuf, acc.at[pl.ds(row0 + i * _ZB, _ZB), :])
            return c

        lax.fori_loop(0, (_NBLK * _RB) // _ZB, zcp, 0)
        plsc.subcore_barrier()

        ib = (ib0, ib1)
        isems = (isem0, isem1)
        rows = (rows0, rows1)
        gsems = (gsem0, gsem1)
        pltpu.sync_copy(ed_hbm.at[start, :, :], ib0)
        pltpu.async_copy(ed_hbm.at[start + 1, :, :], ib1, isem1)
        pltpu.async_copy(u_hbm.at[ib0.at[0]], rows0, gsem0)

        def body(g, carry):
            for b in range(2):
                c = 2 * g + b
                nb = 1 - b

                @pl.when(c + 1 < t)
                def _():
                    pltpu.make_async_copy(ed_hbm.at[start + c + 1, :, :],
                                          ib[nb], isems[nb]).wait()
                    pltpu.async_copy(u_hbm.at[ib[nb].at[0]],
                                     rows[nb], gsems[nb])

                pltpu.make_async_copy(u_hbm.at[ib[b].at[0]],
                                      rows[b], gsems[b]).wait()
                pltpu.sync_copy(rows[b], acc.at[ib[b].at[1]], add=True)

                @pl.when(c + 2 < t)
                def _():
                    pltpu.async_copy(ed_hbm.at[start + c + 2, :, :],
                                     ib[b], isems[b])
            return carry

        lax.fori_loop(0, t // 2, body, 0)
        plsc.subcore_barrier()
        for b in range(_NBLK):
            r = row0 + b * _RB
            pltpu.sync_copy(acc.at[pl.ds(r, _RB), :],
                            out_hbm.at[cid, pl.ds(r, _RB), :])

    return agg_kernel


def _dinv_of(dp_ref):
    return lax.rsqrt(dp_ref[0] + dp_ref[1] + 1.0)


def _tc_scale_body(dp_ref, x_ref, u_ref):
    u_ref[...] = x_ref[...] * _dinv_of(dp_ref)


def _tc_hidden_body(dp_ref, s_ref, u1_ref, w_ref, b_ref, u2_ref):
    dinv = _dinv_of(dp_ref)
    ax = (s_ref[0] + s_ref[1] + u1_ref[...]) * dinv
    h = jnp.dot(ax, w_ref[...], preferred_element_type=jnp.float32) + b_ref[...]
    u2_ref[...] = jnp.maximum(h, 0.0) * dinv


def _tc_out_body(dp_ref, s_ref, u2_ref, wmu_ref, bmu_ref, wls_ref, bls_ref,
                 mu_ref, ls_ref):
    dinv = _dinv_of(dp_ref)
    ah = (s_ref[0] + s_ref[1] + u2_ref[...]) * dinv
    mu_ref[...] = jnp.dot(ah, wmu_ref[...],
                          preferred_element_type=jnp.float32) + bmu_ref[...]
    ls_ref[...] = jnp.dot(ah, wls_ref[...],
                          preferred_element_type=jnp.float32) + bls_ref[...]


def _dp_spec():
    return pl.BlockSpec((_NC, _R, 1), lambda i: (0, i, 0))


def _row_spec(d):
    return pl.BlockSpec((_R, d), lambda i: (i, 0))


def _s_spec():
    return pl.BlockSpec((_NC, _R, _D), lambda i: (0, i, 0))


def _full_spec(shape):
    return pl.BlockSpec(shape, lambda i: tuple(0 for _ in shape))


def kernel(x, edge_index, W1, b1, W_mu, b_mu, W_logstd, b_logstd):
    n, d = x.shape
    e = edge_index.shape[1]
    q = _NW * _CK * 8
    epad = q * (-(-e // q))
    npad_e = epad - e
    nchunk = epad // (_NW * _CK)
    src = jnp.concatenate(
        [edge_index[0], jnp.zeros((npad_e,), edge_index.dtype)])
    pad_dst = n + (jnp.arange(npad_e, dtype=edge_index.dtype) % (_NPAD - n))
    dst = jnp.concatenate([edge_index[1], pad_dst])
    ed = jnp.stack([src.reshape(-1, _CK), dst.reshape(-1, _CK)],
                   axis=1)

    degp = _make_deg(epad)(dst.reshape(_NW, nchunk, _CK))
    dp3 = degp[:, :, None]

    grid = (_NPAD // _R,)
    u1 = pl.pallas_call(
        _tc_scale_body,
        grid=grid,
        in_specs=[_dp_spec(), _row_spec(d)],
        out_specs=_row_spec(d),
        out_shape=jax.ShapeDtypeStruct((n, d), jnp.float32),
    )(dp3, x)

    agg = _make_agg(epad)
    s1 = agg(ed, u1)

    dhid = W1.shape[1]
    u2 = pl.pallas_call(
        _tc_hidden_body,
        grid=grid,
        in_specs=[_dp_spec(), _s_spec(), _row_spec(d),
                  _full_spec(W1.shape), _full_spec((1, dhid))],
        out_specs=_row_spec(dhid),
        out_shape=jax.ShapeDtypeStruct((n, dhid), jnp.float32),
    )(dp3, s1, u1, W1, b1.reshape(1, -1))

    s2 = agg(ed, u2)

    dout = W_mu.shape[1]
    mu, logstd = pl.pallas_call(
        _tc_out_body,
        grid=grid,
        in_specs=[_dp_spec(), _s_spec(), _row_spec(dhid),
                  _full_spec(W_mu.shape), _full_spec((1, dout)),
                  _full_spec(W_logstd.shape), _full_spec((1, dout))],
        out_specs=[_row_spec(dout), _row_spec(dout)],
        out_shape=[jax.ShapeDtypeStruct((n, dout), jnp.float32),
                   jax.ShapeDtypeStruct((n, dout), jnp.float32)],
    )(dp3, s2, u2, W_mu, b_mu.reshape(1, -1), W_logstd, b_logstd.reshape(1, -1))

    return (mu, logstd)


# speedup vs baseline: 1.0532x; 1.0006x over previous
import functools

import jax
import jax.numpy as jnp
from jax import lax
from jax.experimental import pallas as pl
from jax.experimental.pallas import tpu as pltpu
from jax.experimental.pallas import tpu_sc as plsc

_D = 128
_NC = 2
_NS = 16
_NW = _NC * _NS
_CK = 128
_NPAD = 10240
_RB = 128
_NBLK = _NPAD // (_NS * _RB)
_ZB = 32
_R = 512


def _mesh():
    return plsc.VectorSubcoreMesh(core_axis_name="c", subcore_axis_name="s")


@functools.cache
def _make_deg(epad: int):
    ew = epad // _NW
    nchunk = ew // _CK
    grp = 8
    zper = _NPAD // _NS

    @functools.partial(
        pl.kernel,
        out_type=jax.ShapeDtypeStruct((_NC, _NPAD), jnp.float32),
        mesh=_mesh(),
        scratch_types=[
            pltpu.VMEM((nchunk, _CK), jnp.int32),
            pltpu.VMEM((_CK,), jnp.float32),
            pltpu.VMEM((zper,), jnp.float32),
            pltpu.VMEM_SHARED((_NPAD,), jnp.float32),
            pltpu.SemaphoreType.DMA,
        ],
    )
    def deg_kernel(dst_hbm, out_hbm, idx_all, ones_v, zbuf, acc, sem):
        cid = lax.axis_index("c")
        sid = lax.axis_index("s")
        wid = sid * _NC + cid
        pltpu.sync_copy(dst_hbm.at[wid], idx_all)
        ones16 = jnp.full((16,), 1.0, jnp.float32)
        zero16 = jnp.zeros((16,), jnp.float32)
        for j in range(_CK // 16):
            ones_v[pl.ds(j * 16, 16)] = ones16

        def zb(i, c):
            zbuf[pl.ds(i * 16, 16)] = zero16
            return c

        lax.fori_loop(0, zper // 16, zb, 0)
        pltpu.sync_copy(zbuf, acc.at[pl.ds(sid * zper, zper)])
        plsc.subcore_barrier()

        def body(g, carry):
            for b in range(grp):
                c = g * grp + b
                pltpu.async_copy(ones_v, acc.at[idx_all.at[c]], sem, add=True)
            for b in range(grp):
                pltpu.make_async_copy(ones_v, acc.at[idx_all.at[0]], sem).wait()
            return carry

        lax.fori_loop(0, nchunk // grp, body, 0)
        plsc.subcore_barrier()
        pltpu.sync_copy(acc.at[pl.ds(sid * zper, zper)],
                        out_hbm.at[cid, pl.ds(sid * zper, zper)])

    return deg_kernel


@functools.cache
def _make_agg(epad: int):
    tch = epad // _CK
    per_pair = tch // _NS
    f1 = max(2, (int(per_pair * 0.225) // 2) * 2)
    f0 = per_pair - f1

    @functools.partial(
        pl.kernel,
        out_type=jax.ShapeDtypeStruct((_NC, _NPAD, _D), jnp.float32),
        mesh=_mesh(),
        scratch_types=[
            pltpu.VMEM((2, _CK), jnp.int32),
            pltpu.VMEM((2, _CK), jnp.int32),
            pltpu.VMEM((_CK, _D), jnp.float32),
            pltpu.VMEM((_CK, _D), jnp.float32),
            pltpu.VMEM((_ZB, _D), jnp.float32),
            pltpu.VMEM_SHARED((_NPAD, _D), jnp.float32),
            pltpu.SemaphoreType.DMA,
            pltpu.SemaphoreType.DMA,
            pltpu.SemaphoreType.DMA,
            pltpu.SemaphoreType.DMA,
        ],
    )
    def agg_kernel(ed_hbm, u_hbm, out_hbm,
                   ib0, ib1, rows0, rows1, zbuf, acc,
                   gsem0, gsem1, isem0, isem1):
        cid = lax.axis_index("c")
        sid = lax.axis_index("s")
        start = lax.select(cid == 0, sid * f0, _NS * f0 + sid * f1)
        t = lax.select(cid == 0, jnp.int32(f0), jnp.int32(f1))
        zero16 = jnp.zeros((16,), jnp.float32)

        with jax.named_scope("agg_zero"):
            def zb(i, c):
                for j in range(_D // 16):
                    zbuf[i, pl.ds(j * 16, 16)] = zero16
                return c

            lax.fori_loop(0, _ZB, zb, 0)
            row0 = sid * (_NBLK * _RB)

            def zcp(i, c):
                pltpu.sync_copy(zbuf, acc.at[pl.ds(row0 + i * _ZB, _ZB), :])
                return c

            lax.fori_loop(0, (_NBLK * _RB) // _ZB, zcp, 0)
            plsc.subcore_barrier()

        ib = (ib0, ib1)
        isems = (isem0, isem1)
        rows = (rows0, rows1)
        gsems = (gsem0, gsem1)
        def body(g, carry):
            for b in range(2):
                c = 2 * g + b
                nb = 1 - b

                @pl.when(c + 1 < t)
                def _():
                    pltpu.make_async_copy(ed_hbm.at[start + c + 1, :, :],
                                          ib[nb], isems[nb]).wait()
                    pltpu.async_copy(u_hbm.at[ib[nb].at[0]],
                                     rows[nb], gsems[nb])

                pltpu.make_async_copy(u_hbm.at[ib[b].at[0]],
                                      rows[b], gsems[b]).wait()
                pltpu.sync_copy(rows[b], acc.at[ib[b].at[1]], add=True)

                @pl.when(c + 2 < t)
                def _():
                    pltpu.async_copy(ed_hbm.at[start + c + 2, :, :],
                                     ib[b], isems[b])
            return carry

        with jax.named_scope("agg_loop"):
            pltpu.sync_copy(ed_hbm.at[start, :, :], ib0)
            pltpu.async_copy(ed_hbm.at[start + 1, :, :], ib1, isem1)
            pltpu.async_copy(u_hbm.at[ib0.at[0]], rows0, gsem0)
            lax.fori_loop(0, t // 2, body, 0)
            plsc.subcore_barrier()
        with jax.named_scope("agg_copyout"):
            for b in range(_NBLK):
                r = row0 + b * _RB
                pltpu.sync_copy(acc.at[pl.ds(r, _RB), :],
                                out_hbm.at[cid, pl.ds(r, _RB), :])

    return agg_kernel


def _dinv_of(dp_ref):
    return lax.rsqrt(dp_ref[0] + dp_ref[1] + 1.0)


def _tc_scale_body(dp_ref, x_ref, u_ref):
    u_ref[...] = x_ref[...] * _dinv_of(dp_ref)


def _tc_hidden_body(dp_ref, s_ref, u1_ref, w_ref, b_ref, u2_ref):
    dinv = _dinv_of(dp_ref)
    ax = (s_ref[0] + s_ref[1] + u1_ref[...]) * dinv
    h = jnp.dot(ax, w_ref[...], preferred_element_type=jnp.float32) + b_ref[...]
    u2_ref[...] = jnp.maximum(h, 0.0) * dinv


def _tc_out_body(dp_ref, s_ref, u2_ref, wmu_ref, bmu_ref, wls_ref, bls_ref,
                 mu_ref, ls_ref):
    dinv = _dinv_of(dp_ref)
    ah = (s_ref[0] + s_ref[1] + u2_ref[...]) * dinv
    mu_ref[...] = jnp.dot(ah, wmu_ref[...],
                          preferred_element_type=jnp.float32) + bmu_ref[...]
    ls_ref[...] = jnp.dot(ah, wls_ref[...],
                          preferred_element_type=jnp.float32) + bls_ref[...]


def _dp_spec():
    return pl.BlockSpec((_NC, _R, 1), lambda i: (0, i, 0))


def _row_spec(d):
    return pl.BlockSpec((_R, d), lambda i: (i, 0))


def _s_spec():
    return pl.BlockSpec((_NC, _R, _D), lambda i: (0, i, 0))


def _full_spec(shape):
    return pl.BlockSpec(shape, lambda i: tuple(0 for _ in shape))


def kernel(x, edge_index, W1, b1, W_mu, b_mu, W_logstd, b_logstd):
    n, d = x.shape
    e = edge_index.shape[1]
    q = _NW * _CK * 8
    epad = q * (-(-e // q))
    npad_e = epad - e
    nchunk = epad // (_NW * _CK)
    src = jnp.concatenate(
        [edge_index[0], jnp.zeros((npad_e,), edge_index.dtype)])
    pad_dst = n + (jnp.arange(npad_e, dtype=edge_index.dtype) % (_NPAD - n))
    dst = jnp.concatenate([edge_index[1], pad_dst])
    ed = jnp.stack([src.reshape(-1, _CK), dst.reshape(-1, _CK)],
                   axis=1)

    degp = _make_deg(epad)(dst.reshape(_NW, nchunk, _CK))
    dp3 = degp[:, :, None]

    grid = (_NPAD // _R,)
    u1 = pl.pallas_call(
        _tc_scale_body,
        grid=grid,
        in_specs=[_dp_spec(), _row_spec(d)],
        out_specs=_row_spec(d),
        out_shape=jax.ShapeDtypeStruct((n, d), jnp.float32),
    )(dp3, x)

    agg = _make_agg(epad)
    s1 = agg(ed, u1)

    dhid = W1.shape[1]
    u2 = pl.pallas_call(
        _tc_hidden_body,
        grid=grid,
        in_specs=[_dp_spec(), _s_spec(), _row_spec(d),
                  _full_spec(W1.shape), _full_spec((1, dhid))],
        out_specs=_row_spec(dhid),
        out_shape=jax.ShapeDtypeStruct((n, dhid), jnp.float32),
    )(dp3, s1, u1, W1, b1.reshape(1, -1))

    s2 = agg(ed, u2)

    dout = W_mu.shape[1]
    mu, logstd = pl.pallas_call(
        _tc_out_body,
        grid=grid,
        in_specs=[_dp_spec(), _s_spec(), _row_spec(dhid),
                  _full_spec(W_mu.shape), _full_spec((1, dout)),
                  _full_spec(W_logstd.shape), _full_spec((1, dout))],
        out_specs=[_row_spec(dout), _row_spec(dout)],
        out_shape=[jax.ShapeDtypeStruct((n, dout), jnp.float32),
                   jax.ShapeDtypeStruct((n, dout), jnp.float32)],
    )(dp3, s2, u2, W_mu, b_mu.reshape(1, -1), W_logstd, b_logstd.reshape(1, -1))

    return (mu, logstd)
